# baseline (speedup 1.0000x reference)
import functools

import jax
import jax.numpy as jnp
from jax import lax
from jax.experimental import pallas as pl
from jax.experimental.pallas import tpu as pltpu
from jax.experimental.pallas import tpu_sc as plsc

NN = 10000
EE = 320000
DD = 128
CC = 40
EPS = 1e-5
ISQ = float(1.0 / (1.0 + EPS) ** 0.5)
QS = 256.0
IQS = 1.0 / QS

NC = 2
NS = 16
NW = NC * NS
NP = 10240
RPT = NP // NS
EPW = EE // NW
K = 80
NCHUNK = EPW // K

BB = 1000
GRID = NN // BB


def _sc_mesh():
    return plsc.VectorSubcoreMesh(
        core_axis_name="c", subcore_axis_name="s", num_cores=NC, num_subcores=NS
    )


def _sc_segment_pass(feat, src, dst, zeros2, zeros1, ones, with_counts):
    out_type = [jax.ShapeDtypeStruct((NC, NP, DD), jnp.float32)]
    scratch = [
        pltpu.VMEM((K,), jnp.int32),
        pltpu.VMEM((K,), jnp.int32),
        pltpu.VMEM((K,), jnp.int32),
        pltpu.VMEM((K,), jnp.int32),
        pltpu.VMEM((K, DD), jnp.float32),
        pltpu.VMEM((K, DD), jnp.float32),
        pltpu.VMEM_SHARED((NP, DD), jnp.float32),
        pltpu.SemaphoreType.DMA,
        pltpu.SemaphoreType.DMA,
        pltpu.SemaphoreType.DMA,
        pltpu.SemaphoreType.DMA,
        pltpu.SemaphoreType.DMA,
        pltpu.SemaphoreType.DMA,
        pltpu.SemaphoreType.DMA,
        pltpu.SemaphoreType.DMA,
        pltpu.SemaphoreType.DMA,
        pltpu.SemaphoreType.DMA,
        pltpu.SemaphoreType.DMA,
        pltpu.SemaphoreType.DMA,
    ]
    if with_counts:
        out_type.append(jax.ShapeDtypeStruct((NC * NP,), jnp.float32))
        scratch += [
            pltpu.VMEM((K,), jnp.float32),
            pltpu.VMEM_SHARED((NP,), jnp.float32),
        ]

    def body(*refs):
        if with_counts:
            (feat_h, src_h, dst_h, z2_h, z1_h, ones_h, sums_h, cnts_h,
             s_a, s_b, d_a, d_b, rows_a, rows_b, acc,
             sem_ra, sem_rb, sem_sa, sem_sb, sem_da, sem_db,
             sem_wa, sem_wb, sem_ca, sem_cb, sem_r2a, sem_r2b,
             ones_v, cacc) = refs
        else:
            (feat_h, src_h, dst_h, z2_h, sums_h,
             s_a, s_b, d_a, d_b, rows_a, rows_b, acc,
             sem_ra, sem_rb, sem_sa, sem_sb, sem_da, sem_db,
             sem_wa, sem_wb, sem_ca, sem_cb, sem_r2a, sem_r2b) = refs
        sv = [s_a, s_b]
        dv = [d_a, d_b]
        rv = [rows_a, rows_b]
        sem_r = [sem_ra, sem_rb]
        sem_r2 = [sem_r2a, sem_r2b]
        sem_s = [sem_sa, sem_sb]
        sem_d = [sem_da, sem_db]
        sem_w = [sem_wa, sem_wb]
        sem_c = [sem_ca, sem_cb]
        c = lax.axis_index("c")
        s = lax.axis_index("s")
        wid = c * NS + s
        rbase = pl.multiple_of(s * RPT, 8)
        pltpu.sync_copy(z2_h, acc.at[pl.ds(rbase, RPT)])
        if with_counts:
            pltpu.sync_copy(z1_h, cacc.at[pl.ds(rbase, RPT)])
            pltpu.sync_copy(ones_h, ones_v)
        plsc.subcore_barrier()

        ebase = wid * EPW

        def ioff(j):
            return pl.multiple_of(ebase + j * K, 8)

        def clamp(j):
            return jnp.minimum(j, NCHUNK - 1)

        def iload(h, j, buf, sem):
            pltpu.async_copy(h.at[pl.ds(ioff(j), K)], buf, sem)

        def iwait(h, buf, sem):
            pltpu.make_async_copy(h.at[pl.ds(ioff(0), K)], buf, sem).wait()

        def gath(x):
            pltpu.make_async_copy(feat_h.at[sv[x]], rv[x], sem_r[x]).start()

        def gwait(x):
            pltpu.make_async_copy(feat_h.at[sv[x]], rv[x], sem_r[x]).wait()

        def scat_start(x):
            pltpu.make_async_copy(rv[x], acc.at[dv[x]],
                                  sem_w[x]).start(add=True)
            if with_counts:
                pltpu.make_async_copy(ones_v, cacc.at[dv[x]],
                                      sem_c[x]).start(add=True)

        def scat_wait(x):
            pltpu.make_async_copy(rv[x], acc.at[dv[x]], sem_w[x]).wait()
            if with_counts:
                pltpu.make_async_copy(ones_v, cacc.at[dv[x]],
                                      sem_c[x]).wait()

        def half(j, x, first):
            y = 1 - x
            if not first:
                scat_wait(y)
            iwait(src_h, sv[y], sem_s[y])
            gath(y)
            gwait(x)
            iwait(dst_h, dv[x], sem_d[x])
            scat_start(x)
            iload(src_h, clamp(j + 2), sv[x], sem_s[x])
            iload(dst_h, clamp(j + 1), dv[y], sem_d[y])

        pltpu.sync_copy(src_h.at[pl.ds(ioff(0), K)], s_a)
        gath(0)
        iload(dst_h, 0, d_a, sem_da)
        iload(src_h, 1, s_b, sem_sb)
        half(0, 0, True)

        def step(i, carry):
            half(2 * i + 1, 1, False)
            half(2 * i + 2, 0, False)
            return carry

        lax.fori_loop(0, (NCHUNK - 1) // 2, step, 0)
        scat_wait(0)
        gwait(1)
        iwait(src_h, s_a, sem_sa)
        iwait(dst_h, d_b, sem_db)
        plsc.subcore_barrier()
        pltpu.sync_copy(acc.at[pl.ds(rbase, RPT)],
                        sums_h.at[c, pl.ds(rbase, RPT)])
        if with_counts:
            cb = pl.multiple_of(c * NP + rbase, 8)
            pltpu.sync_copy(cacc.at[pl.ds(rbase, RPT)],
                            cnts_h.at[pl.ds(cb, RPT)])

    fn = pl.kernel(body, out_type=tuple(out_type), mesh=_sc_mesh(),
                   scratch_types=scratch)
    if with_counts:
        return fn(feat, src, dst, zeros2, zeros1, ones)
    return fn(feat, src, dst, zeros2)[0]


def _dense1_body(x, pa, cn, wl, wr, wres, g, b, br, c1_o, h_o):
    s1 = pa[0] + pa[1]
    cnt = cn[0] + cn[1]
    rcp = 1.0 / jnp.maximum(cnt, 1.0)
    agg = s1 * rcp
    t = (jnp.dot(agg, wl[...], preferred_element_type=jnp.float32)
         + jnp.dot(x[...], wr[...], preferred_element_type=jnp.float32))
    t = g[...] * (t * ISQ) + b[...]
    c1 = jnp.maximum(t, 0.0)
    c1_o[...] = c1
    h = c1 + jnp.dot(x[...], wres[...],
                     preferred_element_type=jnp.float32) + br[...]
    h_o[...] = h


def _dense2a_body(x, c1, h, wr2, w0x, w0c1, b0, tw_o, zz0_o):
    tw_o[...] = jnp.dot(h[...], wr2[...], preferred_element_type=jnp.float32)
    zz0_o[...] = (jnp.dot(x[...], w0x[...], preferred_element_type=jnp.float32)
                  + jnp.dot(c1[...], w0c1[...],
                            preferred_element_type=jnp.float32)
                  + b0[...])


def _dense2b_body(tw, zz0, pb, cn, wl2, g2, b2,
                  w0c2, gm, bm, w1, b1m, out_o):
    s2 = pb[0] + pb[1]
    cnt = cn[0] + cn[1]
    rcp = 1.0 / jnp.maximum(cnt, 1.0)
    agg2 = s2 * rcp
    t = jnp.dot(agg2, wl2[...], preferred_element_type=jnp.float32) + tw[...]
    t = g2[...] * (t * ISQ) + b2[...]
    c2 = jnp.maximum(t, 0.0)
    zz = (jnp.dot(c2, w0c2[...], preferred_element_type=jnp.float32)
          + zz0[...])
    z1 = jnp.maximum(gm[...] * (zz * ISQ) + bm[...], 0.0)
    full = jnp.dot(z1, w1[...], preferred_element_type=jnp.float32) + b1m[...]
    out_o[...] = full[:, :CC]


def _row_spec(k=DD):
    return pl.BlockSpec((BB, k), lambda i: (i, 0))


def _w_spec():
    return pl.BlockSpec((DD, DD), lambda i: (0, 0))


def _v_spec(k=DD):
    return pl.BlockSpec((1, k), lambda i: (0, 0))


def _pad_cols(a, k=DD):
    return jnp.pad(a, [(0, 0)] * (a.ndim - 1) + [(0, k - a.shape[-1])])


def kernel(x, edge_index, Wl1, Wr1, g1, b1, Wl2, Wr2, g2, b2,
           Wres, bres, Wm0, bm0, gm, bm, Wm1, bm1):
    f32 = jnp.float32
    src = edge_index[0]
    dst = edge_index[1]
    zeros2 = jnp.zeros((RPT, DD), f32)
    zeros1 = jnp.zeros((RPT,), f32)
    ones = jnp.ones((K,), f32)
    sums_a, cnts = _sc_segment_pass(x, src, dst, zeros2, zeros1, ones, True)
    pa = sums_a
    cn = cnts.reshape(NC, NP, 1)

    cn_spec = pl.BlockSpec((NC, BB, 1), lambda i: (0, i, 0))
    pa_spec = pl.BlockSpec((NC, BB, DD), lambda i: (0, i, 0))
    c1, h = pl.pallas_call(
        _dense1_body,
        grid=(GRID,),
        in_specs=[_row_spec(), pa_spec, cn_spec, _w_spec(), _w_spec(),
                  _w_spec(), _v_spec(), _v_spec(), _v_spec()],
        out_specs=(_row_spec(), _row_spec()),
        out_shape=(jax.ShapeDtypeStruct((NN, DD), f32),
                   jax.ShapeDtypeStruct((NN, DD), f32)),
    )(x, pa, cn, Wl1, Wr1, Wres, g1.reshape(1, DD), b1.reshape(1, DD),
      bres.reshape(1, DD))

    pb = _sc_segment_pass(h, src, dst, zeros2, None, None, False)

    w0x = _pad_cols(Wm0[0:DD])
    w0c1 = _pad_cols(Wm0[DD:2 * DD])
    w0c2 = _pad_cols(Wm0[2 * DD:3 * DD])
    b0 = _pad_cols(bm0.reshape(1, -1))
    gmp = _pad_cols(gm.reshape(1, -1))
    bmp = _pad_cols(bm.reshape(1, -1))
    w1 = jnp.pad(Wm1, [(0, DD - Wm1.shape[0]), (0, DD - Wm1.shape[1])])
    b1m = _pad_cols(bm1.reshape(1, -1))
    tw, zz0 = pl.pallas_call(
        _dense2a_body,
        grid=(GRID,),
        in_specs=[_row_spec(), _row_spec(), _row_spec(),
                  _w_spec(), _w_spec(), _w_spec(), _v_spec()],
        out_specs=(_row_spec(), _row_spec()),
        out_shape=(jax.ShapeDtypeStruct((NN, DD), f32),
                   jax.ShapeDtypeStruct((NN, DD), f32)),
    )(x, c1, h, Wr2, w0x, w0c1, b0)

    out = pl.pallas_call(
        _dense2b_body,
        grid=(GRID,),
        in_specs=[_row_spec(), _row_spec(), pa_spec, cn_spec,
                  _w_spec(), _v_spec(), _v_spec(),
                  _w_spec(), _v_spec(), _v_spec(), _w_spec(), _v_spec()],
        out_specs=_row_spec(CC),
        out_shape=jax.ShapeDtypeStruct((NN, CC), f32),
    )(tw, zz0, pb, cn, Wl2, g2.reshape(1, DD), b2.reshape(1, DD),
      w0c2, gmp, bmp, w1, b1m)
    return out


# speedup vs baseline: 1.0001x; 1.0001x over previous
import functools

import jax
import jax.numpy as jnp
from jax import lax
from jax.experimental import pallas as pl
from jax.experimental.pallas import tpu as pltpu
from jax.experimental.pallas import tpu_sc as plsc

NN = 10000
EE = 320000
DD = 128
CC = 40
EPS = 1e-5
ISQ = float(1.0 / (1.0 + EPS) ** 0.5)
QS = 256.0
IQS = 1.0 / QS

NC = 2
NS = 16
NW = NC * NS
NP = 10240
RPT = NP // NS
EPW = EE // NW
K = 80
NCHUNK = EPW // K

BB = 1000
GRID = NN // BB


def _sc_mesh():
    return plsc.VectorSubcoreMesh(
        core_axis_name="c", subcore_axis_name="s", num_cores=NC, num_subcores=NS
    )


def _sc_segment_pass(feat, src, dst, zeros2, zeros1, ones, with_counts):
    out_type = [jax.ShapeDtypeStruct((NC, NP, DD), jnp.float32)]
    scratch = [
        pltpu.VMEM((K,), jnp.int32),
        pltpu.VMEM((K,), jnp.int32),
        pltpu.VMEM((K,), jnp.int32),
        pltpu.VMEM((K,), jnp.int32),
        pltpu.VMEM((K, DD), jnp.float32),
        pltpu.VMEM((K, DD), jnp.float32),
        pltpu.VMEM_SHARED((NP, DD), jnp.float32),
        pltpu.SemaphoreType.DMA,
        pltpu.SemaphoreType.DMA,
        pltpu.SemaphoreType.DMA,
        pltpu.SemaphoreType.DMA,
        pltpu.SemaphoreType.DMA,
        pltpu.SemaphoreType.DMA,
        pltpu.SemaphoreType.DMA,
        pltpu.SemaphoreType.DMA,
        pltpu.SemaphoreType.DMA,
        pltpu.SemaphoreType.DMA,
        pltpu.SemaphoreType.DMA,
        pltpu.SemaphoreType.DMA,
    ]
    if with_counts:
        out_type.append(jax.ShapeDtypeStruct((NC * NP,), jnp.float32))
        scratch += [
            pltpu.VMEM((K,), jnp.float32),
            pltpu.VMEM_SHARED((NP,), jnp.float32),
        ]

    def body(*refs):
        if with_counts:
            (feat_h, src_h, dst_h, z2_h, z1_h, ones_h, sums_h, cnts_h,
             s_a, s_b, d_a, d_b, rows_a, rows_b, acc,
             sem_ra, sem_rb, sem_sa, sem_sb, sem_da, sem_db,
             sem_wa, sem_wb, sem_ca, sem_cb, sem_r2a, sem_r2b,
             ones_v, cacc) = refs
        else:
            (feat_h, src_h, dst_h, z2_h, sums_h,
             s_a, s_b, d_a, d_b, rows_a, rows_b, acc,
             sem_ra, sem_rb, sem_sa, sem_sb, sem_da, sem_db,
             sem_wa, sem_wb, sem_ca, sem_cb, sem_r2a, sem_r2b) = refs
        sv = [s_a, s_b]
        dv = [d_a, d_b]
        rv = [rows_a, rows_b]
        sem_r = [sem_ra, sem_rb]
        sem_r2 = [sem_r2a, sem_r2b]
        sem_s = [sem_sa, sem_sb]
        sem_d = [sem_da, sem_db]
        sem_w = [sem_wa, sem_wb]
        sem_c = [sem_ca, sem_cb]
        c = lax.axis_index("c")
        s = lax.axis_index("s")
        wid = c * NS + s
        rbase = pl.multiple_of(s * RPT, 8)
        pltpu.sync_copy(z2_h, acc.at[pl.ds(rbase, RPT)])
        if with_counts:
            pltpu.sync_copy(z1_h, cacc.at[pl.ds(rbase, RPT)])
            pltpu.sync_copy(ones_h, ones_v)
        plsc.subcore_barrier()

        ebase = wid * EPW

        def ioff(j):
            return pl.multiple_of(ebase + j * K, 8)

        def clamp(j):
            return jnp.minimum(j, NCHUNK - 1)

        def iload(h, j, buf, sem):
            pltpu.async_copy(h.at[pl.ds(ioff(j), K)], buf, sem)

        def iwait(h, buf, sem):
            pltpu.make_async_copy(h.at[pl.ds(ioff(0), K)], buf, sem).wait()

        def gath(x):
            pltpu.make_async_copy(feat_h.at[sv[x]], rv[x], sem_r[x]).start()

        def gwait(x):
            pltpu.make_async_copy(feat_h.at[sv[x]], rv[x], sem_r[x]).wait()

        def scat_start(x):
            pltpu.make_async_copy(rv[x], acc.at[dv[x]],
                                  sem_w[x]).start(add=True)
            if with_counts:
                pltpu.make_async_copy(ones_v, cacc.at[dv[x]],
                                      sem_c[x]).start(add=True)

        def scat_wait(x):
            pltpu.make_async_copy(rv[x], acc.at[dv[x]], sem_w[x]).wait()
            if with_counts:
                pltpu.make_async_copy(ones_v, cacc.at[dv[x]],
                                      sem_c[x]).wait()

        def half(j, x, first):
            y = 1 - x
            if not first:
                scat_wait(y)
            iwait(src_h, sv[y], sem_s[y])
            gath(y)
            gwait(x)
            iwait(dst_h, dv[x], sem_d[x])
            scat_start(x)
            iload(src_h, clamp(j + 2), sv[x], sem_s[x])
            iload(dst_h, clamp(j + 1), dv[y], sem_d[y])

        pltpu.sync_copy(src_h.at[pl.ds(ioff(0), K)], s_a)
        gath(0)
        iload(dst_h, 0, d_a, sem_da)
        iload(src_h, 1, s_b, sem_sb)
        half(0, 0, True)

        def step(i, carry):
            half(2 * i + 1, 1, False)
            half(2 * i + 2, 0, False)
            return carry

        lax.fori_loop(0, (NCHUNK - 1) // 2, step, 0)
        scat_wait(0)
        gwait(1)
        iwait(src_h, s_a, sem_sa)
        iwait(dst_h, d_b, sem_db)
        plsc.subcore_barrier()
        pltpu.sync_copy(acc.at[pl.ds(rbase, RPT)],
                        sums_h.at[c, pl.ds(rbase, RPT)])
        if with_counts:
            cb = pl.multiple_of(c * NP + rbase, 8)
            pltpu.sync_copy(cacc.at[pl.ds(rbase, RPT)],
                            cnts_h.at[pl.ds(cb, RPT)])

    fn = pl.kernel(body, out_type=tuple(out_type), mesh=_sc_mesh(),
                   scratch_types=scratch)
    if with_counts:
        return fn(feat, src, dst, zeros2, zeros1, ones)
    return fn(feat, src, dst, zeros2)[0]


def _dense1_body(x, pa, cn, wl, wr, wres, g, b, br, c1_o, h_o):
    s1 = pa[0] + pa[1]
    cnt = cn[0] + cn[1]
    rcp = 1.0 / jnp.maximum(cnt, 1.0)
    agg = s1 * rcp
    t = (jnp.dot(agg, wl[...], preferred_element_type=jnp.float32)
         + jnp.dot(x[...], wr[...], preferred_element_type=jnp.float32))
    t = g[...] * (t * ISQ) + b[...]
    c1 = jnp.maximum(t, 0.0)
    c1_o[...] = c1
    h = c1 + jnp.dot(x[...], wres[...],
                     preferred_element_type=jnp.float32) + br[...]
    h_o[...] = h


def _dense2a_body(x, c1, h, wr2, w0x, w0c1, b0, tw_o, zz0_o):
    tw_o[...] = jnp.dot(h[...], wr2[...], preferred_element_type=jnp.float32)
    zz0_o[...] = (jnp.dot(x[...], w0x[...], preferred_element_type=jnp.float32)
                  + jnp.dot(c1[...], w0c1[...],
                            preferred_element_type=jnp.float32)
                  + b0[...])


def _dense2b_body(tw, zz0, pb, cn, wl2, g2, b2,
                  w0c2, gm, bm, w1, b1m, out_o):
    s2 = pb[0] + pb[1]
    cnt = cn[0] + cn[1]
    rcp = 1.0 / jnp.maximum(cnt, 1.0)
    agg2 = s2 * rcp
    t = jnp.dot(agg2, wl2[...], preferred_element_type=jnp.float32) + tw[...]
    t = g2[...] * (t * ISQ) + b2[...]
    c2 = jnp.maximum(t, 0.0)
    zz = (jnp.dot(c2, w0c2[...], preferred_element_type=jnp.float32)
          + zz0[...])
    z1 = jnp.maximum(gm[...] * (zz * ISQ) + bm[...], 0.0)
    full = jnp.dot(z1, w1[...], preferred_element_type=jnp.float32) + b1m[...]
    out_o[...] = full[:, :CC]


def _row_spec(k=DD):
    return pl.BlockSpec((BB, k), lambda i: (i, 0))


def _w_spec():
    return pl.BlockSpec((DD, DD), lambda i: (0, 0))


def _v_spec(k=DD):
    return pl.BlockSpec((1, k), lambda i: (0, 0))


def _pad_cols(a, k=DD):
    return jnp.pad(a, [(0, 0)] * (a.ndim - 1) + [(0, k - a.shape[-1])])


def kernel(x, edge_index, Wl1, Wr1, g1, b1, Wl2, Wr2, g2, b2,
           Wres, bres, Wm0, bm0, gm, bm, Wm1, bm1):
    f32 = jnp.float32
    src = edge_index[0]
    dst = edge_index[1]
    zeros2 = jnp.zeros((RPT, DD), f32)
    zeros1 = jnp.zeros((RPT,), f32)
    ones = jnp.ones((K,), f32)
    sums_a, cnts = _sc_segment_pass(x, src, dst, zeros2, zeros1, ones, True)
    pa = sums_a
    cn = cnts.reshape(NC, NP, 1)

    cn_spec = pl.BlockSpec((NC, BB, 1), lambda i: (0, i, 0))
    pa_spec = pl.BlockSpec((NC, BB, DD), lambda i: (0, i, 0))
    c1, h = pl.pallas_call(
        _dense1_body,
        grid=(GRID,),
        in_specs=[_row_spec(), pa_spec, cn_spec, _w_spec(), _w_spec(),
                  _w_spec(), _v_spec(), _v_spec(), _v_spec()],
        out_specs=(_row_spec(), _row_spec()),
        out_shape=(jax.ShapeDtypeStruct((NN, DD), f32),
                   jax.ShapeDtypeStruct((NN, DD), f32)),
    )(x, pa, cn, Wl1, Wr1, Wres, g1.reshape(1, DD), b1.reshape(1, DD),
      bres.reshape(1, DD))

    w0x = _pad_cols(Wm0[0:DD])
    w0c1 = _pad_cols(Wm0[DD:2 * DD])
    w0c2 = _pad_cols(Wm0[2 * DD:3 * DD])
    b0 = _pad_cols(bm0.reshape(1, -1))
    gmp = _pad_cols(gm.reshape(1, -1))
    bmp = _pad_cols(bm.reshape(1, -1))
    w1 = jnp.pad(Wm1, [(0, DD - Wm1.shape[0]), (0, DD - Wm1.shape[1])])
    b1m = _pad_cols(bm1.reshape(1, -1))
    tw, zz0 = pl.pallas_call(
        _dense2a_body,
        grid=(GRID,),
        in_specs=[_row_spec(), _row_spec(), _row_spec(),
                  _w_spec(), _w_spec(), _w_spec(), _v_spec()],
        out_specs=(_row_spec(), _row_spec()),
        out_shape=(jax.ShapeDtypeStruct((NN, DD), f32),
                   jax.ShapeDtypeStruct((NN, DD), f32)),
    )(x, c1, h, Wr2, w0x, w0c1, b0)

    pb = _sc_segment_pass(h, src, dst, zeros2, None, None, False)

    out = pl.pallas_call(
        _dense2b_body,
        grid=(GRID,),
        in_specs=[_row_spec(), _row_spec(), pa_spec, cn_spec,
                  _w_spec(), _v_spec(), _v_spec(),
                  _w_spec(), _v_spec(), _v_spec(), _w_spec(), _v_spec()],
        out_specs=_row_spec(CC),
        out_shape=jax.ShapeDtypeStruct((NN, CC), f32),
    )(tw, zz0, pb, cn, Wl2, g2.reshape(1, DD), b2.reshape(1, DD),
      w0c2, gmp, bmp, w1, b1m)
    return out


# speedup vs baseline: 1.0572x; 1.0571x over previous
import functools

import jax
import jax.numpy as jnp
from jax import lax
from jax.experimental import pallas as pl
from jax.experimental.pallas import tpu as pltpu
from jax.experimental.pallas import tpu_sc as plsc

NN = 10000
EE = 320000
DD = 128
CC = 40
EPS = 1e-5
ISQ = float(1.0 / (1.0 + EPS) ** 0.5)
QS = 256.0
IQS = 1.0 / QS

NC = 2
NS = 16
NW = NC * NS
NP = 10240
RPT = NP // NS
EPW = EE // NW
K = 80
NCHUNK = EPW // K

BB = 1000
GRID = NN // BB


def _sc_mesh():
    return plsc.VectorSubcoreMesh(
        core_axis_name="c", subcore_axis_name="s", num_cores=NC, num_subcores=NS
    )


def _sc_segment_pass(feat, edges, zeros2, zeros1, ones, with_counts):
    out_type = [jax.ShapeDtypeStruct((NC, NP, DD), jnp.float32)]
    scratch = [
        pltpu.VMEM((K,), jnp.int32),
        pltpu.VMEM((K,), jnp.int32),
        pltpu.VMEM((K,), jnp.int32),
        pltpu.VMEM((K,), jnp.int32),
        pltpu.VMEM((K, DD), jnp.float32),
        pltpu.VMEM((K, DD), jnp.float32),
        pltpu.VMEM_SHARED((NP, DD), jnp.float32),
        pltpu.SemaphoreType.DMA,
        pltpu.SemaphoreType.DMA,
        pltpu.SemaphoreType.DMA,
        pltpu.SemaphoreType.DMA,
        pltpu.SemaphoreType.DMA,
        pltpu.SemaphoreType.DMA,
        pltpu.SemaphoreType.DMA,
        pltpu.SemaphoreType.DMA,
        pltpu.SemaphoreType.DMA,
        pltpu.SemaphoreType.DMA,
        pltpu.SemaphoreType.DMA,
        pltpu.SemaphoreType.DMA,
    ]
    if with_counts:
        out_type.append(jax.ShapeDtypeStruct((NC * NP,), jnp.float32))
        scratch += [
            pltpu.VMEM((K,), jnp.float32),
            pltpu.VMEM_SHARED((NP,), jnp.float32),
        ]

    def body(*refs):
        if with_counts:
            (feat_h, edges_h, z2_h, z1_h, ones_h, sums_h, cnts_h,
             s_a, s_b, d_a, d_b, rows_a, rows_b, acc,
             sem_ra, sem_rb, sem_sa, sem_sb, sem_da, sem_db,
             sem_wa, sem_wb, sem_ca, sem_cb, sem_r2a, sem_r2b,
             ones_v, cacc) = refs
        else:
            (feat_h, edges_h, z2_h, sums_h,
             s_a, s_b, d_a, d_b, rows_a, rows_b, acc,
             sem_ra, sem_rb, sem_sa, sem_sb, sem_da, sem_db,
             sem_wa, sem_wb, sem_ca, sem_cb, sem_r2a, sem_r2b) = refs
        sv = [s_a, s_b]
        dv = [d_a, d_b]
        rv = [rows_a, rows_b]
        sem_r = [sem_ra, sem_rb]
        sem_r2 = [sem_r2a, sem_r2b]
        sem_s = [sem_sa, sem_sb]
        sem_d = [sem_da, sem_db]
        sem_w = [sem_wa, sem_wb]
        sem_c = [sem_ca, sem_cb]
        c = lax.axis_index("c")
        s = lax.axis_index("s")
        wid = c * NS + s
        rbase = pl.multiple_of(s * RPT, 8)
        pltpu.sync_copy(z2_h, acc.at[pl.ds(rbase, RPT)])
        if with_counts:
            pltpu.sync_copy(z1_h, cacc.at[pl.ds(rbase, RPT)])
            pltpu.sync_copy(ones_h, ones_v)
        plsc.subcore_barrier()

        ebase = wid * EPW

        def ioff(j):
            return pl.multiple_of(ebase + j * K, 8)

        def clamp(j):
            return jnp.minimum(j, NCHUNK - 1)

        def iload(off0, j, buf, sem):
            pltpu.async_copy(edges_h.at[pl.ds(off0 + ioff(j), K)], buf, sem)

        def iwait(buf, sem):
            pltpu.make_async_copy(edges_h.at[pl.ds(ioff(0), K)], buf,
                                  sem).wait()

        def gath(x):
            pltpu.make_async_copy(feat_h.at[sv[x]], rv[x], sem_r[x]).start()

        def gwait(x):
            pltpu.make_async_copy(feat_h.at[sv[x]], rv[x], sem_r[x]).wait()

        def scat_start(x):
            pltpu.make_async_copy(rv[x], acc.at[dv[x]],
                                  sem_w[x]).start(add=True)
            if with_counts:
                pltpu.make_async_copy(ones_v, cacc.at[dv[x]],
                                      sem_c[x]).start(add=True)

        def scat_wait(x):
            pltpu.make_async_copy(rv[x], acc.at[dv[x]], sem_w[x]).wait()
            if with_counts:
                pltpu.make_async_copy(ones_v, cacc.at[dv[x]],
                                      sem_c[x]).wait()

        def half(j, x, first):
            y = 1 - x
            if not first:
                scat_wait(y)
            iwait(sv[y], sem_s[y])
            gath(y)
            gwait(x)
            iwait(dv[x], sem_d[x])
            scat_start(x)
            iload(0, clamp(j + 2), sv[x], sem_s[x])
            iload(EE, clamp(j + 1), dv[y], sem_d[y])

        pltpu.sync_copy(edges_h.at[pl.ds(ioff(0), K)], s_a)
        gath(0)
        iload(EE, 0, d_a, sem_da)
        iload(0, 1, s_b, sem_sb)
        half(0, 0, True)

        def step(i, carry):
            half(2 * i + 1, 1, False)
            half(2 * i + 2, 0, False)
            return carry

        lax.fori_loop(0, (NCHUNK - 1) // 2, step, 0)
        scat_wait(0)
        gwait(1)
        iwait(s_a, sem_sa)
        iwait(d_b, sem_db)
        plsc.subcore_barrier()
        pltpu.sync_copy(acc.at[pl.ds(rbase, RPT)],
                        sums_h.at[c, pl.ds(rbase, RPT)])
        if with_counts:
            cb = pl.multiple_of(c * NP + rbase, 8)
            pltpu.sync_copy(cacc.at[pl.ds(rbase, RPT)],
                            cnts_h.at[pl.ds(cb, RPT)])

    fn = pl.kernel(body, out_type=tuple(out_type), mesh=_sc_mesh(),
                   scratch_types=scratch)
    if with_counts:
        return fn(feat, edges, zeros2, zeros1, ones)
    return fn(feat, edges, zeros2)[0]


def _dense1_body(x, pa, rcp, wl, wr, wres, g, b, br, c1_o, h_o):
    s1 = pa[0] + pa[1]
    agg = s1 * rcp[...]
    t = (jnp.dot(agg, wl[...], preferred_element_type=jnp.float32)
         + jnp.dot(x[...], wr[...], preferred_element_type=jnp.float32))
    t = g[...] * (t * ISQ) + b[...]
    c1 = jnp.maximum(t, 0.0)
    c1_o[...] = c1
    h = c1 + jnp.dot(x[...], wres[...],
                     preferred_element_type=jnp.float32) + br[...]
    h_o[...] = h


def _dense2a_body(x, c1, h, wr2, w0x, w0c1, b0, tw_o, zz0_o):
    tw_o[...] = jnp.dot(h[...], wr2[...], preferred_element_type=jnp.float32)
    zz0_o[...] = (jnp.dot(x[...], w0x[...], preferred_element_type=jnp.float32)
                  + jnp.dot(c1[...], w0c1[...],
                            preferred_element_type=jnp.float32)
                  + b0[...])


def _dense2b_body(tw, zz0, pb, rcp, wl2, g2, b2,
                  w0c2, gm, bm, w1, b1m, out_o):
    s2 = pb[0] + pb[1]
    agg2 = s2 * rcp[...]
    t = jnp.dot(agg2, wl2[...], preferred_element_type=jnp.float32) + tw[...]
    t = g2[...] * (t * ISQ) + b2[...]
    c2 = jnp.maximum(t, 0.0)
    zz = (jnp.dot(c2, w0c2[...], preferred_element_type=jnp.float32)
          + zz0[...])
    z1 = jnp.maximum(gm[...] * (zz * ISQ) + bm[...], 0.0)
    full = jnp.dot(z1, w1[...], preferred_element_type=jnp.float32) + b1m[...]
    out_o[...] = full[:, :CC]


def _row_spec(k=DD):
    return pl.BlockSpec((BB, k), lambda i: (i, 0))


def _w_spec():
    return pl.BlockSpec((DD, DD), lambda i: (0, 0))


def _v_spec(k=DD):
    return pl.BlockSpec((1, k), lambda i: (0, 0))


def _pad_cols(a, k=DD):
    return jnp.pad(a, [(0, 0)] * (a.ndim - 1) + [(0, k - a.shape[-1])])


def kernel(x, edge_index, Wl1, Wr1, g1, b1, Wl2, Wr2, g2, b2,
           Wres, bres, Wm0, bm0, gm, bm, Wm1, bm1):
    f32 = jnp.float32
    edges = edge_index.reshape(2 * EE)
    zeros2 = jnp.zeros((RPT, DD), f32)
    zeros1 = jnp.zeros((RPT,), f32)
    ones = jnp.ones((K,), f32)
    sums_a, cnts = _sc_segment_pass(x, edges, zeros2, zeros1, ones, True)
    pa = sums_a
    cnt2 = cnts.reshape(NC, NP)
    rcp = (1.0 / jnp.maximum(cnt2[0] + cnt2[1], 1.0)).reshape(NP, 1)

    rcp_spec = pl.BlockSpec((BB, 1), lambda i: (i, 0))
    pa_spec = pl.BlockSpec((NC, BB, DD), lambda i: (0, i, 0))
    c1, h = pl.pallas_call(
        _dense1_body,
        grid=(GRID,),
        in_specs=[_row_spec(), pa_spec, rcp_spec, _w_spec(), _w_spec(),
                  _w_spec(), _v_spec(), _v_spec(), _v_spec()],
        out_specs=(_row_spec(), _row_spec()),
        out_shape=(jax.ShapeDtypeStruct((NN, DD), f32),
                   jax.ShapeDtypeStruct((NN, DD), f32)),
    )(x, pa, rcp, Wl1, Wr1, Wres, g1.reshape(1, DD), b1.reshape(1, DD),
      bres.reshape(1, DD))

    w0x = _pad_cols(Wm0[0:DD])
    w0c1 = _pad_cols(Wm0[DD:2 * DD])
    w0c2 = _pad_cols(Wm0[2 * DD:3 * DD])
    b0 = _pad_cols(bm0.reshape(1, -1))
    gmp = _pad_cols(gm.reshape(1, -1))
    bmp = _pad_cols(bm.reshape(1, -1))
    w1 = jnp.pad(Wm1, [(0, DD - Wm1.shape[0]), (0, DD - Wm1.shape[1])])
    b1m = _pad_cols(bm1.reshape(1, -1))
    tw, zz0 = pl.pallas_call(
        _dense2a_body,
        grid=(GRID,),
        in_specs=[_row_spec(), _row_spec(), _row_spec(),
                  _w_spec(), _w_spec(), _w_spec(), _v_spec()],
        out_specs=(_row_spec(), _row_spec()),
        out_shape=(jax.ShapeDtypeStruct((NN, DD), f32),
                   jax.ShapeDtypeStruct((NN, DD), f32)),
    )(x, c1, h, Wr2, w0x, w0c1, b0)

    pb = _sc_segment_pass(h, edges, zeros2, None, None, False)

    out = pl.pallas_call(
        _dense2b_body,
        grid=(GRID,),
        in_specs=[_row_spec(), _row_spec(), pa_spec, rcp_spec,
                  _w_spec(), _v_spec(), _v_spec(),
                  _w_spec(), _v_spec(), _v_spec(), _w_spec(), _v_spec()],
        out_specs=_row_spec(CC),
        out_shape=jax.ShapeDtypeStruct((NN, CC), f32),
    )(tw, zz0, pb, rcp, Wl2, g2.reshape(1, DD), b2.reshape(1, DD),
      w0c2, gmp, bmp, w1, b1m)
    return out


# speedup vs baseline: 1.0781x; 1.0198x over previous
import functools

import jax
import jax.numpy as jnp
from jax import lax
from jax.experimental import pallas as pl
from jax.experimental.pallas import tpu as pltpu
from jax.experimental.pallas import tpu_sc as plsc

NN = 10000
EE = 320000
DD = 128
CC = 40
EPS = 1e-5
ISQ = float(1.0 / (1.0 + EPS) ** 0.5)
QS = 256.0
IQS = 1.0 / QS

NC = 2
NS = 16
NW = NC * NS
NP = 10240
RPT = NP // NS
EPW = EE // NW
K = 80
NCHUNK = EPW // K

BB = 2000
GRID = NN // BB


def _sc_mesh():
    return plsc.VectorSubcoreMesh(
        core_axis_name="c", subcore_axis_name="s", num_cores=NC, num_subcores=NS
    )


def _sc_segment_pass(feat, edges, zeros2, zeros1, ones, with_counts):
    out_type = [jax.ShapeDtypeStruct((NC, NP, DD), jnp.float32)]
    scratch = [
        pltpu.VMEM((K,), jnp.int32),
        pltpu.VMEM((K,), jnp.int32),
        pltpu.VMEM((K,), jnp.int32),
        pltpu.VMEM((K,), jnp.int32),
        pltpu.VMEM((K, DD), jnp.float32),
        pltpu.VMEM((K, DD), jnp.float32),
        pltpu.VMEM_SHARED((NP, DD), jnp.float32),
        pltpu.SemaphoreType.DMA,
        pltpu.SemaphoreType.DMA,
        pltpu.SemaphoreType.DMA,
        pltpu.SemaphoreType.DMA,
        pltpu.SemaphoreType.DMA,
        pltpu.SemaphoreType.DMA,
        pltpu.SemaphoreType.DMA,
        pltpu.SemaphoreType.DMA,
        pltpu.SemaphoreType.DMA,
        pltpu.SemaphoreType.DMA,
        pltpu.SemaphoreType.DMA,
        pltpu.SemaphoreType.DMA,
    ]
    if with_counts:
        out_type.append(jax.ShapeDtypeStruct((NC * NP,), jnp.float32))
        scratch += [
            pltpu.VMEM((K,), jnp.float32),
            pltpu.VMEM_SHARED((NP,), jnp.float32),
        ]

    def body(*refs):
        if with_counts:
            (feat_h, edges_h, z2_h, z1_h, ones_h, sums_h, cnts_h,
             s_a, s_b, d_a, d_b, rows_a, rows_b, acc,
             sem_ra, sem_rb, sem_sa, sem_sb, sem_da, sem_db,
             sem_wa, sem_wb, sem_ca, sem_cb, sem_r2a, sem_r2b,
             ones_v, cacc) = refs
        else:
            (feat_h, edges_h, z2_h, sums_h,
             s_a, s_b, d_a, d_b, rows_a, rows_b, acc,
             sem_ra, sem_rb, sem_sa, sem_sb, sem_da, sem_db,
             sem_wa, sem_wb, sem_ca, sem_cb, sem_r2a, sem_r2b) = refs
        sv = [s_a, s_b]
        dv = [d_a, d_b]
        rv = [rows_a, rows_b]
        sem_r = [sem_ra, sem_rb]
        sem_r2 = [sem_r2a, sem_r2b]
        sem_s = [sem_sa, sem_sb]
        sem_d = [sem_da, sem_db]
        sem_w = [sem_wa, sem_wb]
        sem_c = [sem_ca, sem_cb]
        c = lax.axis_index("c")
        s = lax.axis_index("s")
        wid = c * NS + s
        rbase = pl.multiple_of(s * RPT, 8)
        ebase = wid * EPW

        def ioff(j):
            return pl.multiple_of(ebase + j * K, 8)

        def clamp(j):
            return jnp.minimum(j, NCHUNK - 1)

        def iload(off0, j, buf, sem):
            pltpu.async_copy(edges_h.at[pl.ds(off0 + ioff(j), K)], buf, sem)

        def iwait(buf, sem):
            pltpu.make_async_copy(edges_h.at[pl.ds(ioff(0), K)], buf,
                                  sem).wait()

        def gath(x):
            pltpu.make_async_copy(feat_h.at[sv[x]], rv[x], sem_r[x]).start()

        def gwait(x):
            pltpu.make_async_copy(feat_h.at[sv[x]], rv[x], sem_r[x]).wait()

        def scat_start(x):
            pltpu.make_async_copy(rv[x], acc.at[dv[x]],
                                  sem_w[x]).start(add=True)
            if with_counts:
                pltpu.make_async_copy(ones_v, cacc.at[dv[x]],
                                      sem_c[x]).start(add=True)

        def scat_wait(x):
            pltpu.make_async_copy(rv[x], acc.at[dv[x]], sem_w[x]).wait()
            if with_counts:
                pltpu.make_async_copy(ones_v, cacc.at[dv[x]],
                                      sem_c[x]).wait()

        def half(j, x, first):
            y = 1 - x
            if not first:
                scat_wait(y)
            iwait(sv[y], sem_s[y])
            gath(y)
            gwait(x)
            iwait(dv[x], sem_d[x])
            scat_start(x)
            iload(0, clamp(j + 2), sv[x], sem_s[x])
            iload(EE, clamp(j + 1), dv[y], sem_d[y])

        iload(0, 0, s_a, sem_sa)
        pltpu.sync_copy(z2_h, acc.at[pl.ds(rbase, RPT)])
        if with_counts:
            pltpu.sync_copy(z1_h, cacc.at[pl.ds(rbase, RPT)])
            pltpu.sync_copy(ones_h, ones_v)
        iwait(s_a, sem_sa)
        gath(0)
        iload(EE, 0, d_a, sem_da)
        iload(0, 1, s_b, sem_sb)
        plsc.subcore_barrier()
        half(0, 0, True)

        def step(i, carry):
            half(2 * i + 1, 1, False)
            half(2 * i + 2, 0, False)
            return carry

        lax.fori_loop(0, (NCHUNK - 1) // 2, step, 0)
        scat_wait(0)
        gwait(1)
        iwait(s_a, sem_sa)
        iwait(d_b, sem_db)
        plsc.subcore_barrier()
        pltpu.sync_copy(acc.at[pl.ds(rbase, RPT)],
                        sums_h.at[c, pl.ds(rbase, RPT)])
        if with_counts:
            cb = pl.multiple_of(c * NP + rbase, 8)
            pltpu.sync_copy(cacc.at[pl.ds(rbase, RPT)],
                            cnts_h.at[pl.ds(cb, RPT)])

    fn = pl.kernel(body, out_type=tuple(out_type), mesh=_sc_mesh(),
                   scratch_types=scratch)
    if with_counts:
        return fn(feat, edges, zeros2, zeros1, ones)
    return fn(feat, edges, zeros2)[0]


def _dense1_body(x, pa, rcp, wl, wr, wres, g, b, br, c1_o, h_o):
    s1 = pa[0] + pa[1]
    agg = s1 * rcp[...]
    t = (jnp.dot(agg, wl[...], preferred_element_type=jnp.float32)
         + jnp.dot(x[...], wr[...], preferred_element_type=jnp.float32))
    t = g[...] * (t * ISQ) + b[...]
    c1 = jnp.maximum(t, 0.0)
    c1_o[...] = c1
    h = c1 + jnp.dot(x[...], wres[...],
                     preferred_element_type=jnp.float32) + br[...]
    h_o[...] = h


def _dense2a_body(x, c1, h, wr2, w0x, w0c1, b0, tw_o, zz0_o):
    tw_o[...] = jnp.dot(h[...], wr2[...], preferred_element_type=jnp.float32)
    zz0_o[...] = (jnp.dot(x[...], w0x[...], preferred_element_type=jnp.float32)
                  + jnp.dot(c1[...], w0c1[...],
                            preferred_element_type=jnp.float32)
                  + b0[...])


def _dense2b_body(tw, zz0, pb, rcp, wl2, g2, b2,
                  w0c2, gm, bm, w1, b1m, out_o):
    s2 = pb[0] + pb[1]
    agg2 = s2 * rcp[...]
    t = jnp.dot(agg2, wl2[...], preferred_element_type=jnp.float32) + tw[...]
    t = g2[...] * (t * ISQ) + b2[...]
    c2 = jnp.maximum(t, 0.0)
    zz = (jnp.dot(c2, w0c2[...], preferred_element_type=jnp.float32)
          + zz0[...])
    z1 = jnp.maximum(gm[...] * (zz * ISQ) + bm[...], 0.0)
    full = jnp.dot(z1, w1[...], preferred_element_type=jnp.float32) + b1m[...]
    out_o[...] = full[:, :CC]


def _row_spec(k=DD):
    return pl.BlockSpec((BB, k), lambda i: (i, 0))


def _w_spec():
    return pl.BlockSpec((DD, DD), lambda i: (0, 0))


def _v_spec(k=DD):
    return pl.BlockSpec((1, k), lambda i: (0, 0))


def _pad_cols(a, k=DD):
    return jnp.pad(a, [(0, 0)] * (a.ndim - 1) + [(0, k - a.shape[-1])])


def kernel(x, edge_index, Wl1, Wr1, g1, b1, Wl2, Wr2, g2, b2,
           Wres, bres, Wm0, bm0, gm, bm, Wm1, bm1):
    f32 = jnp.float32
    edges = edge_index.reshape(2 * EE)
    zeros2 = jnp.zeros((RPT, DD), f32)
    zeros1 = jnp.zeros((RPT,), f32)
    ones = jnp.ones((K,), f32)
    sums_a, cnts = _sc_segment_pass(x, edges, zeros2, zeros1, ones, True)
    pa = sums_a
    cnt2 = cnts.reshape(NC, NP)
    rcp = (1.0 / jnp.maximum(cnt2[0] + cnt2[1], 1.0)).reshape(NP, 1)

    rcp_spec = pl.BlockSpec((BB, 1), lambda i: (i, 0))
    pa_spec = pl.BlockSpec((NC, BB, DD), lambda i: (0, i, 0))
    c1, h = pl.pallas_call(
        _dense1_body,
        grid=(GRID,),
        in_specs=[_row_spec(), pa_spec, rcp_spec, _w_spec(), _w_spec(),
                  _w_spec(), _v_spec(), _v_spec(), _v_spec()],
        out_specs=(_row_spec(), _row_spec()),
        out_shape=(jax.ShapeDtypeStruct((NN, DD), f32),
                   jax.ShapeDtypeStruct((NN, DD), f32)),
    )(x, pa, rcp, Wl1, Wr1, Wres, g1.reshape(1, DD), b1.reshape(1, DD),
      bres.reshape(1, DD))

    w0x = _pad_cols(Wm0[0:DD])
    w0c1 = _pad_cols(Wm0[DD:2 * DD])
    w0c2 = _pad_cols(Wm0[2 * DD:3 * DD])
    b0 = _pad_cols(bm0.reshape(1, -1))
    gmp = _pad_cols(gm.reshape(1, -1))
    bmp = _pad_cols(bm.reshape(1, -1))
    w1 = jnp.pad(Wm1, [(0, DD - Wm1.shape[0]), (0, DD - Wm1.shape[1])])
    b1m = _pad_cols(bm1.reshape(1, -1))
    tw, zz0 = pl.pallas_call(
        _dense2a_body,
        grid=(GRID,),
        in_specs=[_row_spec(), _row_spec(), _row_spec(),
                  _w_spec(), _w_spec(), _w_spec(), _v_spec()],
        out_specs=(_row_spec(), _row_spec()),
        out_shape=(jax.ShapeDtypeStruct((NN, DD), f32),
                   jax.ShapeDtypeStruct((NN, DD), f32)),
    )(x, c1, h, Wr2, w0x, w0c1, b0)

    pb = _sc_segment_pass(h, edges, zeros2, None, None, False)

    out = pl.pallas_call(
        _dense2b_body,
        grid=(GRID,),
        in_specs=[_row_spec(), _row_spec(), pa_spec, rcp_spec,
                  _w_spec(), _v_spec(), _v_spec(),
                  _w_spec(), _v_spec(), _v_spec(), _w_spec(), _v_spec()],
        out_specs=_row_spec(CC),
        out_shape=jax.ShapeDtypeStruct((NN, CC), f32),
    )(tw, zz0, pb, rcp, Wl2, g2.reshape(1, DD), b2.reshape(1, DD),
      w0c2, gmp, bmp, w1, b1m)
    return out


# speedup vs baseline: 1.2678x; 1.1759x over previous
import functools

import jax
import jax.numpy as jnp
from jax import lax
from jax.experimental import pallas as pl
from jax.experimental.pallas import tpu as pltpu
from jax.experimental.pallas import tpu_sc as plsc

NN = 10000
EE = 320000
DD = 128
CC = 40
EPS = 1e-5
ISQ = float(1.0 / (1.0 + EPS) ** 0.5)
QS = 256.0
IQS = 1.0 / QS

NC = 2
NS = 16
NW = NC * NS
NP = 10240
RPT = NP // NS
EPW = EE // NW
K = 80
NCHUNK = EPW // K

BB = 2000
GRID = NN // BB


def _sc_mesh():
    return plsc.VectorSubcoreMesh(
        core_axis_name="c", subcore_axis_name="s", num_cores=NC, num_subcores=NS
    )


def _sc_segment_pass(feat, edges, zeros2, zeros1, ones, with_counts):
    out_type = [jax.ShapeDtypeStruct((NC, NP, DD), jnp.float32)]
    scratch = (
        [pltpu.VMEM((K,), jnp.int32)] * 3
        + [pltpu.VMEM((K,), jnp.int32)] * 3
        + [pltpu.VMEM((K, DD), jnp.float32)] * 3
        + [pltpu.VMEM_SHARED((NP, DD), jnp.float32)]
        + [pltpu.SemaphoreType.DMA] * 15
    )
    if with_counts:
        out_type.append(jax.ShapeDtypeStruct((NC * NP,), jnp.float32))
        scratch += [
            pltpu.VMEM((K,), jnp.float32),
            pltpu.VMEM_SHARED((NP,), jnp.float32),
        ]

    def body(*refs):
        if with_counts:
            hbm, rest = refs[:7], refs[7:]
            (feat_h, edges_h, z2_h, z1_h, ones_h, sums_h, cnts_h) = hbm
            ones_v, cacc = rest[-2:]
            rest = rest[:-2]
        else:
            hbm, rest = refs[:4], refs[4:]
            (feat_h, edges_h, z2_h, sums_h) = hbm
        sv = list(rest[0:3])
        dv = list(rest[3:6])
        rv = list(rest[6:9])
        acc = rest[9]
        sems = rest[10:25]
        sem_r = list(sems[0:3])
        sem_s = list(sems[3:6])
        sem_d = list(sems[6:9])
        sem_w = list(sems[9:12])
        sem_c = list(sems[12:15])
        c = lax.axis_index("c")
        s = lax.axis_index("s")
        wid = c * NS + s
        rbase = pl.multiple_of(s * RPT, 8)
        ebase = wid * EPW

        def ioff(j):
            return pl.multiple_of(ebase + j * K, 8)

        def clamp(j):
            return jnp.minimum(j, NCHUNK - 1)

        def iload(off0, j, buf, sem):
            pltpu.async_copy(edges_h.at[pl.ds(off0 + ioff(j), K)], buf, sem)

        def iwait(buf, sem):
            pltpu.make_async_copy(edges_h.at[pl.ds(ioff(0), K)], buf,
                                  sem).wait()

        def gath(x):
            pltpu.make_async_copy(feat_h.at[sv[x]], rv[x], sem_r[x]).start()

        def gwait(x):
            pltpu.make_async_copy(feat_h.at[sv[x]], rv[x], sem_r[x]).wait()

        def scat_start(x):
            pltpu.make_async_copy(rv[x], acc.at[dv[x]],
                                  sem_w[x]).start(add=True)
            if with_counts:
                pltpu.make_async_copy(ones_v, cacc.at[dv[x]],
                                      sem_c[x]).start(add=True)

        def scat_wait(x):
            pltpu.make_async_copy(rv[x], acc.at[dv[x]], sem_w[x]).wait()
            if with_counts:
                pltpu.make_async_copy(ones_v, cacc.at[dv[x]],
                                      sem_c[x]).wait()

        def half(j, x, first):
            y = (x + 1) % 3
            z = (x + 2) % 3
            if not first:
                scat_wait(z)
            iwait(sv[z], sem_s[z])
            gath(z)
            gwait(x)
            iwait(dv[x], sem_d[x])
            scat_start(x)
            iload(0, clamp(j + 3), sv[x], sem_s[x])
            iload(EE, clamp(j + 2), dv[z], sem_d[z])

        iload(0, 0, sv[0], sem_s[0])
        pltpu.sync_copy(z2_h, acc.at[pl.ds(rbase, RPT)])
        if with_counts:
            pltpu.sync_copy(z1_h, cacc.at[pl.ds(rbase, RPT)])
            pltpu.sync_copy(ones_h, ones_v)
        iwait(sv[0], sem_s[0])
        gath(0)
        iload(0, 1, sv[1], sem_s[1])
        iload(EE, 0, dv[0], sem_d[0])
        iload(EE, 1, dv[1], sem_d[1])
        iload(0, 2, sv[2], sem_s[2])
        iwait(sv[1], sem_s[1])
        gath(1)
        plsc.subcore_barrier()
        half(0, 0, True)

        def step(i, carry):
            half(3 * i + 1, 1, False)
            half(3 * i + 2, 2, False)
            half(3 * i + 3, 0, False)
            return carry

        lax.fori_loop(0, (NCHUNK - 2) // 3, step, 0)
        xl = (NCHUNK - 1) % 3
        scat_wait((xl + 2) % 3)
        gwait(xl)
        iwait(dv[xl], sem_d[xl])
        scat_start(xl)
        scat_wait(xl)
        gwait((xl + 1) % 3)
        iwait(sv[(xl + 2) % 3], sem_s[(xl + 2) % 3])
        iwait(dv[(xl + 1) % 3], sem_d[(xl + 1) % 3])
        plsc.subcore_barrier()
        pltpu.sync_copy(acc.at[pl.ds(rbase, RPT)],
                        sums_h.at[c, pl.ds(rbase, RPT)])
        if with_counts:
            cb = pl.multiple_of(c * NP + rbase, 8)
            pltpu.sync_copy(cacc.at[pl.ds(rbase, RPT)],
                            cnts_h.at[pl.ds(cb, RPT)])

    fn = pl.kernel(body, out_type=tuple(out_type), mesh=_sc_mesh(),
                   scratch_types=scratch)
    if with_counts:
        return fn(feat, edges, zeros2, zeros1, ones)
    return fn(feat, edges, zeros2)[0]


def _dense1_body(x, pa, rcp, wl, wr, wres, g, b, br, c1_o, h_o):
    s1 = pa[0] + pa[1]
    agg = s1 * rcp[...]
    t = (jnp.dot(agg, wl[...], preferred_element_type=jnp.float32)
         + jnp.dot(x[...], wr[...], preferred_element_type=jnp.float32))
    t = g[...] * (t * ISQ) + b[...]
    c1 = jnp.maximum(t, 0.0)
    c1_o[...] = c1
    h = c1 + jnp.dot(x[...], wres[...],
                     preferred_element_type=jnp.float32) + br[...]
    h_o[...] = h


def _dense2a_body(x, c1, h, wr2, w0x, w0c1, b0, tw_o, zz0_o):
    tw_o[...] = jnp.dot(h[...], wr2[...], preferred_element_type=jnp.float32)
    zz0_o[...] = (jnp.dot(x[...], w0x[...], preferred_element_type=jnp.float32)
                  + jnp.dot(c1[...], w0c1[...],
                            preferred_element_type=jnp.float32)
                  + b0[...])


def _dense2b_body(tw, zz0, pb, rcp, wl2, g2, b2,
                  w0c2, gm, bm, w1, b1m, out_o):
    s2 = pb[0] + pb[1]
    agg2 = s2 * rcp[...]
    t = jnp.dot(agg2, wl2[...], preferred_element_type=jnp.float32) + tw[...]
    t = g2[...] * (t * ISQ) + b2[...]
    c2 = jnp.maximum(t, 0.0)
    zz = (jnp.dot(c2, w0c2[...], preferred_element_type=jnp.float32)
          + zz0[...])
    z1 = jnp.maximum(gm[...] * (zz * ISQ) + bm[...], 0.0)
    full = jnp.dot(z1, w1[...], preferred_element_type=jnp.float32) + b1m[...]
    out_o[...] = full[:, :CC]


def _row_spec(k=DD):
    return pl.BlockSpec((BB, k), lambda i: (i, 0))


def _w_spec():
    return pl.BlockSpec((DD, DD), lambda i: (0, 0))


def _v_spec(k=DD):
    return pl.BlockSpec((1, k), lambda i: (0, 0))


def _pad_cols(a, k=DD):
    return jnp.pad(a, [(0, 0)] * (a.ndim - 1) + [(0, k - a.shape[-1])])


def kernel(x, edge_index, Wl1, Wr1, g1, b1, Wl2, Wr2, g2, b2,
           Wres, bres, Wm0, bm0, gm, bm, Wm1, bm1):
    f32 = jnp.float32
    edges = edge_index.reshape(2 * EE)
    zeros2 = jnp.zeros((RPT, DD), f32)
    zeros1 = jnp.zeros((RPT,), f32)
    ones = jnp.ones((K,), f32)
    sums_a, cnts = _sc_segment_pass(x, edges, zeros2, zeros1, ones, True)
    pa = sums_a
    cnt2 = cnts.reshape(NC, NP)
    rcp = (1.0 / jnp.maximum(cnt2[0] + cnt2[1], 1.0)).reshape(NP, 1)

    rcp_spec = pl.BlockSpec((BB, 1), lambda i: (i, 0))
    pa_spec = pl.BlockSpec((NC, BB, DD), lambda i: (0, i, 0))
    c1, h = pl.pallas_call(
        _dense1_body,
        grid=(GRID,),
        in_specs=[_row_spec(), pa_spec, rcp_spec, _w_spec(), _w_spec(),
                  _w_spec(), _v_spec(), _v_spec(), _v_spec()],
        out_specs=(_row_spec(), _row_spec()),
        out_shape=(jax.ShapeDtypeStruct((NN, DD), f32),
                   jax.ShapeDtypeStruct((NN, DD), f32)),
    )(x, pa, rcp, Wl1, Wr1, Wres, g1.reshape(1, DD), b1.reshape(1, DD),
      bres.reshape(1, DD))

    w0x = _pad_cols(Wm0[0:DD])
    w0c1 = _pad_cols(Wm0[DD:2 * DD])
    w0c2 = _pad_cols(Wm0[2 * DD:3 * DD])
    b0 = _pad_cols(bm0.reshape(1, -1))
    gmp = _pad_cols(gm.reshape(1, -1))
    bmp = _pad_cols(bm.reshape(1, -1))
    w1 = jnp.pad(Wm1, [(0, DD - Wm1.shape[0]), (0, DD - Wm1.shape[1])])
    b1m = _pad_cols(bm1.reshape(1, -1))
    tw, zz0 = pl.pallas_call(
        _dense2a_body,
        grid=(GRID,),
        in_specs=[_row_spec(), _row_spec(), _row_spec(),
                  _w_spec(), _w_spec(), _w_spec(), _v_spec()],
        out_specs=(_row_spec(), _row_spec()),
        out_shape=(jax.ShapeDtypeStruct((NN, DD), f32),
                   jax.ShapeDtypeStruct((NN, DD), f32)),
    )(x, c1, h, Wr2, w0x, w0c1, b0)

    pb = _sc_segment_pass(h, edges, zeros2, None, None, False)

    out = pl.pallas_call(
        _dense2b_body,
        grid=(GRID,),
        in_specs=[_row_spec(), _row_spec(), pa_spec, rcp_spec,
                  _w_spec(), _v_spec(), _v_spec(),
                  _w_spec(), _v_spec(), _v_spec(), _w_spec(), _v_spec()],
        out_specs=_row_spec(CC),
        out_shape=jax.ShapeDtypeStruct((NN, CC), f32),
    )(tw, zz0, pb, rcp, Wl2, g2.reshape(1, DD), b2.reshape(1, DD),
      w0c2, gmp, bmp, w1, b1m)
    return out


# speedup vs baseline: 1.3048x; 1.0292x over previous
import functools

import jax
import jax.numpy as jnp
from jax import lax
from jax.experimental import pallas as pl
from jax.experimental.pallas import tpu as pltpu
from jax.experimental.pallas import tpu_sc as plsc

NN = 10000
EE = 320000
DD = 128
CC = 40
EPS = 1e-5
ISQ = float(1.0 / (1.0 + EPS) ** 0.5)
QS = 256.0
IQS = 1.0 / QS

NC = 2
NS = 16
NW = NC * NS
NP = 10240
RPT = NP // NS
EPW = EE // NW
K = 80
NCHUNK = EPW // K

BB = 2000
GRID = NN // BB


def _sc_mesh():
    return plsc.VectorSubcoreMesh(
        core_axis_name="c", subcore_axis_name="s", num_cores=NC, num_subcores=NS
    )


def _sc_segment_pass(feat, edges, zeros2, zeros1, ones, with_counts):
    out_type = [jax.ShapeDtypeStruct((NC, NP, DD), jnp.float32)]
    scratch = (
        [pltpu.VMEM((K,), jnp.int32)] * 4
        + [pltpu.VMEM((K,), jnp.int32)] * 4
        + [pltpu.VMEM((K, DD), jnp.float32)] * 4
        + [pltpu.VMEM_SHARED((NP, DD), jnp.float32)]
        + [pltpu.SemaphoreType.DMA] * 20
    )
    if with_counts:
        out_type.append(jax.ShapeDtypeStruct((NC * NP,), jnp.float32))
        scratch += [
            pltpu.VMEM((K,), jnp.float32),
            pltpu.VMEM_SHARED((NP,), jnp.float32),
        ]

    def body(*refs):
        if with_counts:
            hbm, rest = refs[:7], refs[7:]
            (feat_h, edges_h, z2_h, z1_h, ones_h, sums_h, cnts_h) = hbm
            ones_v, cacc = rest[-2:]
            rest = rest[:-2]
        else:
            hbm, rest = refs[:4], refs[4:]
            (feat_h, edges_h, z2_h, sums_h) = hbm
        sv = list(rest[0:4])
        dv = list(rest[4:8])
        rv = list(rest[8:12])
        acc = rest[12]
        sems = rest[13:33]
        sem_r = list(sems[0:4])
        sem_s = list(sems[4:8])
        sem_d = list(sems[8:12])
        sem_w = list(sems[12:16])
        sem_c = list(sems[16:20])
        c = lax.axis_index("c")
        s = lax.axis_index("s")
        wid = c * NS + s
        rbase = pl.multiple_of(s * RPT, 8)
        ebase = wid * EPW

        def ioff(j):
            return pl.multiple_of(ebase + j * K, 8)

        def clamp(j):
            return jnp.minimum(j, NCHUNK - 1)

        def iload(off0, j, buf, sem):
            pltpu.async_copy(edges_h.at[pl.ds(off0 + ioff(j), K)], buf, sem)

        def iwait(buf, sem):
            pltpu.make_async_copy(edges_h.at[pl.ds(ioff(0), K)], buf,
                                  sem).wait()

        def gath(x):
            pltpu.make_async_copy(feat_h.at[sv[x]], rv[x], sem_r[x]).start()

        def gwait(x):
            pltpu.make_async_copy(feat_h.at[sv[x]], rv[x], sem_r[x]).wait()

        def scat_start(x):
            pltpu.make_async_copy(rv[x], acc.at[dv[x]],
                                  sem_w[x]).start(add=True)
            if with_counts:
                pltpu.make_async_copy(ones_v, cacc.at[dv[x]],
                                      sem_c[x]).start(add=True)

        def scat_wait(x):
            pltpu.make_async_copy(rv[x], acc.at[dv[x]], sem_w[x]).wait()
            if with_counts:
                pltpu.make_async_copy(ones_v, cacc.at[dv[x]],
                                      sem_c[x]).wait()

        def half(j, x, first):
            z2 = (x + 2) % 4
            z3 = (x + 3) % 4
            iwait(sv[z2], sem_s[z2])
            gath(z2)
            gwait(x)
            if not first:
                scat_wait(z3)
            iwait(dv[x], sem_d[x])
            scat_start(x)
            iload(0, clamp(j + 4), sv[x], sem_s[x])
            iload(EE, clamp(j + 3), dv[z3], sem_d[z3])

        iload(0, 0, sv[0], sem_s[0])
        pltpu.sync_copy(z2_h, acc.at[pl.ds(rbase, RPT)])
        if with_counts:
            pltpu.sync_copy(z1_h, cacc.at[pl.ds(rbase, RPT)])
            pltpu.sync_copy(ones_h, ones_v)
        iwait(sv[0], sem_s[0])
        gath(0)
        iload(0, 1, sv[1], sem_s[1])
        iload(EE, 0, dv[0], sem_d[0])
        iload(EE, 1, dv[1], sem_d[1])
        iload(EE, 2, dv[2], sem_d[2])
        iload(0, 2, sv[2], sem_s[2])
        iload(0, 3, sv[3], sem_s[3])
        iwait(sv[1], sem_s[1])
        gath(1)
        plsc.subcore_barrier()
        half(0, 0, True)

        def step(i, carry):
            half(4 * i + 1, 1, False)
            half(4 * i + 2, 2, False)
            half(4 * i + 3, 3, False)
            half(4 * i + 4, 0, False)
            return carry

        lax.fori_loop(0, (NCHUNK - 1) // 4, step, 0)
        xl = (NCHUNK - 1) % 4
        scat_wait(xl)
        gwait((xl + 1) % 4)
        gwait((xl + 2) % 4)
        iwait(sv[(xl + 3) % 4], sem_s[(xl + 3) % 4])
        iwait(sv[xl], sem_s[xl])
        iwait(dv[(xl + 1) % 4], sem_d[(xl + 1) % 4])
        iwait(dv[(xl + 2) % 4], sem_d[(xl + 2) % 4])
        iwait(dv[(xl + 3) % 4], sem_d[(xl + 3) % 4])
        plsc.subcore_barrier()
        pltpu.sync_copy(acc.at[pl.ds(rbase, RPT)],
                        sums_h.at[c, pl.ds(rbase, RPT)])
        if with_counts:
            cb = pl.multiple_of(c * NP + rbase, 8)
            pltpu.sync_copy(cacc.at[pl.ds(rbase, RPT)],
                            cnts_h.at[pl.ds(cb, RPT)])

    fn = pl.kernel(body, out_type=tuple(out_type), mesh=_sc_mesh(),
                   scratch_types=scratch)
    if with_counts:
        return fn(feat, edges, zeros2, zeros1, ones)
    return fn(feat, edges, zeros2)[0]


def _dense1_body(x, pa, rcp, wl, wr, wres, g, b, br, c1_o, h_o):
    s1 = pa[0] + pa[1]
    agg = s1 * rcp[...]
    t = (jnp.dot(agg, wl[...], preferred_element_type=jnp.float32)
         + jnp.dot(x[...], wr[...], preferred_element_type=jnp.float32))
    t = g[...] * (t * ISQ) + b[...]
    c1 = jnp.maximum(t, 0.0)
    c1_o[...] = c1
    h = c1 + jnp.dot(x[...], wres[...],
                     preferred_element_type=jnp.float32) + br[...]
    h_o[...] = h


def _dense2a_body(x, c1, h, wr2, w0x, w0c1, b0, tw_o, zz0_o):
    tw_o[...] = jnp.dot(h[...], wr2[...], preferred_element_type=jnp.float32)
    zz0_o[...] = (jnp.dot(x[...], w0x[...], preferred_element_type=jnp.float32)
                  + jnp.dot(c1[...], w0c1[...],
                            preferred_element_type=jnp.float32)
                  + b0[...])


def _dense2b_body(tw, zz0, pb, rcp, wl2, g2, b2,
                  w0c2, gm, bm, w1, b1m, out_o):
    s2 = pb[0] + pb[1]
    agg2 = s2 * rcp[...]
    t = jnp.dot(agg2, wl2[...], preferred_element_type=jnp.float32) + tw[...]
    t = g2[...] * (t * ISQ) + b2[...]
    c2 = jnp.maximum(t, 0.0)
    zz = (jnp.dot(c2, w0c2[...], preferred_element_type=jnp.float32)
          + zz0[...])
    z1 = jnp.maximum(gm[...] * (zz * ISQ) + bm[...], 0.0)
    full = jnp.dot(z1, w1[...], preferred_element_type=jnp.float32) + b1m[...]
    out_o[...] = full[:, :CC]


def _row_spec(k=DD):
    return pl.BlockSpec((BB, k), lambda i: (i, 0))


def _w_spec():
    return pl.BlockSpec((DD, DD), lambda i: (0, 0))


def _v_spec(k=DD):
    return pl.BlockSpec((1, k), lambda i: (0, 0))


def _pad_cols(a, k=DD):
    return jnp.pad(a, [(0, 0)] * (a.ndim - 1) + [(0, k - a.shape[-1])])


def kernel(x, edge_index, Wl1, Wr1, g1, b1, Wl2, Wr2, g2, b2,
           Wres, bres, Wm0, bm0, gm, bm, Wm1, bm1):
    f32 = jnp.float32
    edges = edge_index.reshape(2 * EE)
    zeros2 = jnp.zeros((RPT, DD), f32)
    zeros1 = jnp.zeros((RPT,), f32)
    ones = jnp.ones((K,), f32)
    sums_a, cnts = _sc_segment_pass(x, edges, zeros2, zeros1, ones, True)
    pa = sums_a
    cnt2 = cnts.reshape(NC, NP)
    rcp = (1.0 / jnp.maximum(cnt2[0] + cnt2[1], 1.0)).reshape(NP, 1)

    rcp_spec = pl.BlockSpec((BB, 1), lambda i: (i, 0))
    pa_spec = pl.BlockSpec((NC, BB, DD), lambda i: (0, i, 0))
    c1, h = pl.pallas_call(
        _dense1_body,
        grid=(GRID,),
        in_specs=[_row_spec(), pa_spec, rcp_spec, _w_spec(), _w_spec(),
                  _w_spec(), _v_spec(), _v_spec(), _v_spec()],
        out_specs=(_row_spec(), _row_spec()),
        out_shape=(jax.ShapeDtypeStruct((NN, DD), f32),
                   jax.ShapeDtypeStruct((NN, DD), f32)),
    )(x, pa, rcp, Wl1, Wr1, Wres, g1.reshape(1, DD), b1.reshape(1, DD),
      bres.reshape(1, DD))

    w0x = _pad_cols(Wm0[0:DD])
    w0c1 = _pad_cols(Wm0[DD:2 * DD])
    w0c2 = _pad_cols(Wm0[2 * DD:3 * DD])
    b0 = _pad_cols(bm0.reshape(1, -1))
    gmp = _pad_cols(gm.reshape(1, -1))
    bmp = _pad_cols(bm.reshape(1, -1))
    w1 = jnp.pad(Wm1, [(0, DD - Wm1.shape[0]), (0, DD - Wm1.shape[1])])
    b1m = _pad_cols(bm1.reshape(1, -1))
    tw, zz0 = pl.pallas_call(
        _dense2a_body,
        grid=(GRID,),
        in_specs=[_row_spec(), _row_spec(), _row_spec(),
                  _w_spec(), _w_spec(), _w_spec(), _v_spec()],
        out_specs=(_row_spec(), _row_spec()),
        out_shape=(jax.ShapeDtypeStruct((NN, DD), f32),
                   jax.ShapeDtypeStruct((NN, DD), f32)),
    )(x, c1, h, Wr2, w0x, w0c1, b0)

    pb = _sc_segment_pass(h, edges, zeros2, None, None, False)

    out = pl.pallas_call(
        _dense2b_body,
        grid=(GRID,),
        in_specs=[_row_spec(), _row_spec(), pa_spec, rcp_spec,
                  _w_spec(), _v_spec(), _v_spec(),
                  _w_spec(), _v_spec(), _v_spec(), _w_spec(), _v_spec()],
        out_specs=_row_spec(CC),
        out_shape=jax.ShapeDtypeStruct((NN, CC), f32),
    )(tw, zz0, pb, rcp, Wl2, g2.reshape(1, DD), b2.reshape(1, DD),
      w0c2, gmp, bmp, w1, b1m)
    return out


# speedup vs baseline: 1.3068x; 1.0015x over previous
import jax
import jax.numpy as jnp
from jax import lax
from jax.experimental import pallas as pl
from jax.experimental.pallas import tpu as pltpu
from jax.experimental.pallas import tpu_sc as plsc

NN = 10000
EE = 320000
DD = 128
CC = 40
EPS = 1e-5
ISQ = float(1.0 / (1.0 + EPS) ** 0.5)

NC = 2
NS = 16
NW = NC * NS
NP = 10240
RPT = NP // NS
EPW = EE // NW
K = 80
NCHUNK = EPW // K

BB = 2000
GRID = NN // BB


def _sc_mesh():
    return plsc.VectorSubcoreMesh(
        core_axis_name="c", subcore_axis_name="s", num_cores=NC, num_subcores=NS
    )


def _sc_segment_pass(feat, edges, zeros2, zeros1, ones, with_counts):
    out_type = [jax.ShapeDtypeStruct((NC, NP, DD), jnp.float32)]
    scratch = (
        [pltpu.VMEM((K,), jnp.int32)] * 4
        + [pltpu.VMEM((K,), jnp.int32)] * 4
        + [pltpu.VMEM((K, DD), jnp.float32)] * 4
        + [pltpu.VMEM_SHARED((NP, DD), jnp.float32)]
        + [pltpu.SemaphoreType.DMA] * 20
    )
    if with_counts:
        out_type.append(jax.ShapeDtypeStruct((NC * NP,), jnp.float32))
        scratch += [
            pltpu.VMEM((K,), jnp.float32),
            pltpu.VMEM_SHARED((NP,), jnp.float32),
        ]

    def body(*refs):
        if with_counts:
            hbm, rest = refs[:7], refs[7:]
            (feat_h, edges_h, z2_h, z1_h, ones_h, sums_h, cnts_h) = hbm
            ones_v, cacc = rest[-2:]
            rest = rest[:-2]
        else:
            hbm, rest = refs[:4], refs[4:]
            (feat_h, edges_h, z2_h, sums_h) = hbm
        sv = list(rest[0:4])
        dv = list(rest[4:8])
        rv = list(rest[8:12])
        acc = rest[12]
        sems = rest[13:33]
        sem_r = list(sems[0:4])
        sem_s = list(sems[4:8])
        sem_d = list(sems[8:12])
        sem_w = list(sems[12:16])
        sem_c = list(sems[16:20])
        c = lax.axis_index("c")
        s = lax.axis_index("s")
        wid = c * NS + s
        rbase = pl.multiple_of(s * RPT, 8)
        ebase = wid * EPW

        def ioff(j):
            return pl.multiple_of(ebase + j * K, 8)

        def clamp(j):
            return jnp.minimum(j, NCHUNK - 1)

        def iload(off0, j, buf, sem):
            pltpu.async_copy(edges_h.at[pl.ds(off0 + ioff(j), K)], buf, sem)

        def iwait(buf, sem):
            pltpu.make_async_copy(edges_h.at[pl.ds(ioff(0), K)], buf,
                                  sem).wait()

        def gath(x):
            pltpu.make_async_copy(feat_h.at[sv[x]], rv[x], sem_r[x]).start()

        def gwait(x):
            pltpu.make_async_copy(feat_h.at[sv[x]], rv[x], sem_r[x]).wait()

        def scat_start(x):
            pltpu.make_async_copy(rv[x], acc.at[dv[x]],
                                  sem_w[x]).start(add=True)
            if with_counts:
                pltpu.make_async_copy(ones_v, cacc.at[dv[x]],
                                      sem_c[x]).start(add=True)

        def scat_wait(x):
            pltpu.make_async_copy(rv[x], acc.at[dv[x]], sem_w[x]).wait()
            if with_counts:
                pltpu.make_async_copy(ones_v, cacc.at[dv[x]],
                                      sem_c[x]).wait()

        def half(j, x, first):
            z2 = (x + 2) % 4
            z3 = (x + 3) % 4
            iwait(sv[z2], sem_s[z2])
            gath(z2)
            gwait(x)
            if not first:
                scat_wait(z3)
            iwait(dv[x], sem_d[x])
            scat_start(x)
            iload(0, clamp(j + 4), sv[x], sem_s[x])
            iload(EE, clamp(j + 3), dv[z3], sem_d[z3])

        iload(0, 0, sv[0], sem_s[0])
        pltpu.sync_copy(z2_h, acc.at[pl.ds(rbase, RPT)])
        if with_counts:
            pltpu.sync_copy(z1_h, cacc.at[pl.ds(rbase, RPT)])
            pltpu.sync_copy(ones_h, ones_v)
        iwait(sv[0], sem_s[0])
        gath(0)
        iload(0, 1, sv[1], sem_s[1])
        iload(EE, 0, dv[0], sem_d[0])
        iload(EE, 1, dv[1], sem_d[1])
        iload(EE, 2, dv[2], sem_d[2])
        iload(0, 2, sv[2], sem_s[2])
        iload(0, 3, sv[3], sem_s[3])
        iwait(sv[1], sem_s[1])
        gath(1)
        plsc.subcore_barrier()
        half(0, 0, True)

        def step(i, carry):
            half(4 * i + 1, 1, False)
            half(4 * i + 2, 2, False)
            half(4 * i + 3, 3, False)
            half(4 * i + 4, 0, False)
            return carry

        lax.fori_loop(0, (NCHUNK - 1) // 4, step, 0)
        xl = (NCHUNK - 1) % 4
        scat_wait(xl)
        gwait((xl + 1) % 4)
        gwait((xl + 2) % 4)
        iwait(sv[(xl + 3) % 4], sem_s[(xl + 3) % 4])
        iwait(sv[xl], sem_s[xl])
        iwait(dv[(xl + 1) % 4], sem_d[(xl + 1) % 4])
        iwait(dv[(xl + 2) % 4], sem_d[(xl + 2) % 4])
        iwait(dv[(xl + 3) % 4], sem_d[(xl + 3) % 4])
        plsc.subcore_barrier()
        pltpu.sync_copy(acc.at[pl.ds(rbase, RPT)],
                        sums_h.at[c, pl.ds(rbase, RPT)])
        if with_counts:
            cb = pl.multiple_of(c * NP + rbase, 8)
            pltpu.sync_copy(cacc.at[pl.ds(rbase, RPT)],
                            cnts_h.at[pl.ds(cb, RPT)])

    fn = pl.kernel(body, out_type=tuple(out_type), mesh=_sc_mesh(),
                   scratch_types=scratch)
    if with_counts:
        return fn(feat, edges, zeros2, zeros1, ones)
    return fn(feat, edges, zeros2)[0]


def _dense1_body(x, pa, rcp, wl, wr, wres, g, b, br, c1_o, h_o):
    s1 = pa[0] + pa[1]
    agg = s1 * rcp[...]
    t = (jnp.dot(agg, wl[...], preferred_element_type=jnp.float32)
         + jnp.dot(x[...], wr[...], preferred_element_type=jnp.float32))
    t = g[...] * (t * ISQ) + b[...]
    c1 = jnp.maximum(t, 0.0)
    c1_o[...] = c1
    h = c1 + jnp.dot(x[...], wres[...],
                     preferred_element_type=jnp.float32) + br[...]
    h_o[...] = h


def _dense2a_body(x, c1, h, wr2, w0x, w0c1, b0, tw_o, zz0_o):
    tw_o[...] = jnp.dot(h[...], wr2[...], preferred_element_type=jnp.float32)
    zz0_o[...] = (jnp.dot(x[...], w0x[...], preferred_element_type=jnp.float32)
                  + jnp.dot(c1[...], w0c1[...],
                            preferred_element_type=jnp.float32)
                  + b0[...])


def _dense2b_body(tw, zz0, pb, rcp, wl2, g2, b2,
                  w0c2, gm, bm, w1, b1m, out_o):
    s2 = pb[0] + pb[1]
    agg2 = s2 * rcp[...]
    t = jnp.dot(agg2, wl2[...], preferred_element_type=jnp.float32) + tw[...]
    t = g2[...] * (t * ISQ) + b2[...]
    c2 = jnp.maximum(t, 0.0)
    zz = (jnp.dot(c2, w0c2[...], preferred_element_type=jnp.float32)
          + zz0[...])
    z1 = jnp.maximum(gm[...] * (zz * ISQ) + bm[...], 0.0)
    full = jnp.dot(z1, w1[...], preferred_element_type=jnp.float32) + b1m[...]
    out_o[...] = full[:, :CC]


def _row_spec(k=DD):
    return pl.BlockSpec((BB, k), lambda i: (i, 0))


def _w_spec():
    return pl.BlockSpec((DD, DD), lambda i: (0, 0))


def _v_spec(k=DD):
    return pl.BlockSpec((1, k), lambda i: (0, 0))


def _pad_cols(a, k=DD):
    return jnp.pad(a, [(0, 0)] * (a.ndim - 1) + [(0, k - a.shape[-1])])


def kernel(x, edge_index, Wl1, Wr1, g1, b1, Wl2, Wr2, g2, b2,
           Wres, bres, Wm0, bm0, gm, bm, Wm1, bm1):
    f32 = jnp.float32
    edges = edge_index.reshape(2 * EE)
    zeros2 = jnp.zeros((RPT, DD), f32)
    zeros1 = jnp.zeros((RPT,), f32)
    ones = jnp.ones((K,), f32)
    sums_a, cnts = _sc_segment_pass(x, edges, zeros2, zeros1, ones, True)
    pa = sums_a
    cnt2 = cnts.reshape(NC, NP)
    rcp = (1.0 / jnp.maximum(cnt2[0] + cnt2[1], 1.0)).reshape(NP, 1)

    rcp_spec = pl.BlockSpec((BB, 1), lambda i: (i, 0))
    pa_spec = pl.BlockSpec((NC, BB, DD), lambda i: (0, i, 0))
    c1, h = pl.pallas_call(
        _dense1_body,
        grid=(GRID,),
        in_specs=[_row_spec(), pa_spec, rcp_spec, _w_spec(), _w_spec(),
                  _w_spec(), _v_spec(), _v_spec(), _v_spec()],
        out_specs=(_row_spec(), _row_spec()),
        out_shape=(jax.ShapeDtypeStruct((NN, DD), f32),
                   jax.ShapeDtypeStruct((NN, DD), f32)),
    )(x, pa, rcp, Wl1, Wr1, Wres, g1.reshape(1, DD), b1.reshape(1, DD),
      bres.reshape(1, DD))

    w0x = _pad_cols(Wm0[0:DD])
    w0c1 = _pad_cols(Wm0[DD:2 * DD])
    w0c2 = _pad_cols(Wm0[2 * DD:3 * DD])
    b0 = _pad_cols(bm0.reshape(1, -1))
    gmp = _pad_cols(gm.reshape(1, -1))
    bmp = _pad_cols(bm.reshape(1, -1))
    w1 = jnp.pad(Wm1, [(0, DD - Wm1.shape[0]), (0, DD - Wm1.shape[1])])
    b1m = _pad_cols(bm1.reshape(1, -1))
    tw, zz0 = pl.pallas_call(
        _dense2a_body,
        grid=(GRID,),
        in_specs=[_row_spec(), _row_spec(), _row_spec(),
                  _w_spec(), _w_spec(), _w_spec(), _v_spec()],
        out_specs=(_row_spec(), _row_spec()),
        out_shape=(jax.ShapeDtypeStruct((NN, DD), f32),
                   jax.ShapeDtypeStruct((NN, DD), f32)),
    )(x, c1, h, Wr2, w0x, w0c1, b0)

    pb = _sc_segment_pass(h, edges, zeros2, None, None, False)

    out = pl.pallas_call(
        _dense2b_body,
        grid=(GRID,),
        in_specs=[_row_spec(), _row_spec(), pa_spec, rcp_spec,
                  _w_spec(), _v_spec(), _v_spec(),
                  _w_spec(), _v_spec(), _v_spec(), _w_spec(), _v_spec()],
        out_specs=_row_spec(CC),
        out_shape=jax.ShapeDtypeStruct((NN, CC), f32),
    )(tw, zz0, pb, rcp, Wl2, g2.reshape(1, DD), b2.reshape(1, DD),
      w0c2, gmp, bmp, w1, b1m)
    return out
